# Initial kernel scaffold; baseline (speedup 1.0000x reference)
import jax
import jax.numpy as jnp
from jax.experimental import pallas as pl


def kernel(y_true, y_pred):
    raise NotImplementedError("write your pallas kernel here")


# speedup vs baseline: 63.7678x; 63.7678x over previous
import functools

import jax
import jax.numpy as jnp
from jax import lax
from jax.experimental import pallas as pl
from jax.experimental.pallas import tpu as pltpu
from jax.experimental.pallas import tpu_sc as plsc

_B = 2048
_W = 2.0 / _B
_N = 16 * 512 * 512
_NW = 32
_PER_TILE = _N // _NW
_CHUNK = 16384
_NCHUNK = _PER_TILE // _CHUNK
_VEC = 16


def _sc_hist_body(yt_hbm, yp_hbm, out_hbm, yt_buf, yp_buf, hist, histr):
    cid = lax.axis_index("c")
    sid = lax.axis_index("s")
    wid = sid * 2 + cid
    base = wid * _PER_TILE

    def zero_body(i, carry):
        hist[pl.ds(i * _VEC, _VEC)] = jnp.zeros((_VEC,), jnp.float32)
        return carry

    lax.fori_loop(0, (32 * _B) // _VEC, zero_body, 0)

    planes = lax.iota(jnp.int32, _VEC) * _B
    ones = jnp.ones((_VEC,), jnp.float32)

    def chunk_body(k, carry):
        off = base + k * _CHUNK
        pltpu.sync_copy(yt_hbm.at[pl.ds(off, _CHUNK)], yt_buf)
        pltpu.sync_copy(yp_hbm.at[pl.ds(off, _CHUNK)], yp_buf)

        def vec_body(i, c2):
            yt = yt_buf[pl.ds(i * _VEC, _VEC)]
            yp = yp_buf[pl.ds(i * _VEC, _VEC)]
            scaled = (yp + 1.0) * (_B / 2.0) - (yp * yt) * float(_B)
            bidx = jnp.clip(scaled.astype(jnp.int32), 0, _B - 1)
            idx = planes + bidx
            plsc.addupdate_scatter(hist, [idx], ones)
            plsc.addupdate_scatter(hist, [idx + 16 * _B], yt)
            return c2

        lax.fori_loop(0, _CHUNK // _VEC, vec_body, 0)
        return carry

    lax.fori_loop(0, _NCHUNK, chunk_body, 0)

    def red_body(j, carry):
        acc_n = jnp.zeros((_VEC,), jnp.float32)
        acc_c = jnp.zeros((_VEC,), jnp.float32)
        for p in range(16):
            acc_n = acc_n + hist[pl.ds(p * _B + j * _VEC, _VEC)]
            acc_c = acc_c + hist[pl.ds((16 + p) * _B + j * _VEC, _VEC)]
        histr[pl.ds(j * _VEC, _VEC)] = acc_n
        histr[pl.ds(_B + j * _VEC, _VEC)] = acc_c
        return carry

    lax.fori_loop(0, _B // _VEC, red_body, 0)
    pltpu.sync_copy(histr, out_hbm.at[wid])


_sc_hist = functools.partial(
    pl.kernel,
    out_type=jax.ShapeDtypeStruct((_NW, 2 * _B), jnp.float32),
    mesh=plsc.VectorSubcoreMesh(core_axis_name="c", subcore_axis_name="s"),
    compiler_params=pltpu.CompilerParams(needs_layout_passes=False),
    scratch_types=[
        pltpu.VMEM((_CHUNK,), jnp.float32),
        pltpu.VMEM((_CHUNK,), jnp.float32),
        pltpu.VMEM((32 * _B,), jnp.float32),
        pltpu.VMEM((2 * _B,), jnp.float32),
    ],
)(_sc_hist_body)


_ROWS = 4096
_COLS = 1024
_BLK = 512


def _bce_body(yt_ref, yp_ref, acc_ref):
    yt = yt_ref[...]
    yp = yp_ref[...]
    logp = jnp.maximum(jnp.log(yp), -100.0)
    logq = jnp.maximum(jnp.log(1.0 - yp), -100.0)
    s = jnp.sum(yt * logp + logq - yt * logq)

    @pl.when(pl.program_id(0) == 0)
    def _init():
        acc_ref[0, 0] = 0.0

    acc_ref[0, 0] += s


def _bce_sum(yt2d, yp2d):
    return pl.pallas_call(
        _bce_body,
        grid=(_ROWS // _BLK,),
        in_specs=[
            pl.BlockSpec((_BLK, _COLS), lambda i: (i, 0)),
            pl.BlockSpec((_BLK, _COLS), lambda i: (i, 0)),
        ],
        out_specs=pl.BlockSpec(memory_space=pltpu.SMEM),
        out_shape=jax.ShapeDtypeStruct((1, 1), jnp.float32),
    )(yt2d, yp2d)


def _finish_body(hist_ref, bce_ref, out_ref):
    h = hist_ref[...].astype(jnp.float32)
    hn = jnp.sum(h[:, :_B], axis=0, keepdims=True)
    hc = jnp.sum(h[:, _B:], axis=0, keepdims=True)
    s1 = jnp.sum(hc)
    s2 = float(_N) - s1
    row = lax.broadcasted_iota(jnp.int32, (_B, _B), 0)
    col = lax.broadcasted_iota(jnp.int32, (_B, _B), 1)
    tri = (row >= col).astype(jnp.float32)
    ncum = jnp.dot(hn, tri, preferred_element_type=jnp.float32,
                   precision=lax.Precision.HIGHEST)
    c1 = jnp.dot(hc, tri, preferred_element_type=jnp.float32,
                 precision=lax.Precision.HIGHEST)
    c2 = ncum - c1

    def jsum(s, c):
        inter = s - c
        union = s + ncum - c
        iz = (union == 0.0).astype(jnp.float32)
        return jnp.sum(1.0 - (inter + iz) / jnp.maximum(union, 1.0))

    loss1 = _W * jsum(s1, c1) - _W * 0.5
    loss2 = _W * jsum(s2, c2) - _W * 0.5
    bce = -bce_ref[0, 0] / float(_N)
    out_ref[0, 0] = (loss1 + loss2) * 0.5 + bce


def _finish(hist, bce):
    return pl.pallas_call(
        _finish_body,
        in_specs=[
            pl.BlockSpec(memory_space=pltpu.VMEM),
            pl.BlockSpec(memory_space=pltpu.SMEM),
        ],
        out_specs=pl.BlockSpec(memory_space=pltpu.SMEM),
        out_shape=jax.ShapeDtypeStruct((1, 1), jnp.float32),
    )(hist, bce)


def kernel(y_true, y_pred):
    yt = y_true.reshape(-1)
    yp = y_pred.reshape(-1)
    hist = _sc_hist(yt, yp)
    bce = _bce_sum(y_true.reshape(_ROWS, _COLS), y_pred.reshape(_ROWS, _COLS))
    out = _finish(hist, bce)
    return out[0, 0]


# speedup vs baseline: 73.4126x; 1.1512x over previous
import functools

import jax
import jax.numpy as jnp
from jax import lax
from jax.experimental import pallas as pl
from jax.experimental.pallas import tpu as pltpu
from jax.experimental.pallas import tpu_sc as plsc

_B = 2048
_W = 2.0 / _B
_N = 16 * 512 * 512
_NW = 32
_PER_TILE = _N // _NW
_CHUNK = 8192
_NCHUNK = _PER_TILE // _CHUNK
_VEC = 16
_U = 8


def _sc_hist_body(yt_hbm, yp_hbm, out_hbm,
                  yt0, yp0, yt1, yp1, hist, histr, sem0, sem1, sem2, sem3):
    cid = lax.axis_index("c")
    sid = lax.axis_index("s")
    wid = sid * 2 + cid
    base = wid * _PER_TILE

    bufs = [(yt0, yp0, sem0, sem1), (yt1, yp1, sem2, sem3)]

    def copies(k, slot):
        off = base + k * _CHUNK
        yt_b, yp_b, sa, sb = bufs[slot]
        return (pltpu.make_async_copy(yt_hbm.at[pl.ds(off, _CHUNK)], yt_b, sa),
                pltpu.make_async_copy(yp_hbm.at[pl.ds(off, _CHUNK)], yp_b, sb))

    for cp in copies(0, 0):
        cp.start()

    def zero_body(i, carry):
        for u in range(16):
            hist[pl.ds(i * 16 * _VEC + u * _VEC, _VEC)] = (
                jnp.zeros((_VEC,), jnp.float32))
        return carry

    lax.fori_loop(0, (32 * _B) // (16 * _VEC), zero_body, 0)

    planes = lax.iota(jnp.int32, _VEC) * _B
    ones = jnp.ones((_VEC,), jnp.float32)

    for k in range(_NCHUNK):
        slot = k % 2
        if k + 1 < _NCHUNK:
            for cp in copies(k + 1, 1 - slot):
                cp.start()
        for cp in copies(k, slot):
            cp.wait()
        yt_b, yp_b = bufs[slot][0], bufs[slot][1]

        def vec_body(i, carry, yt_b=yt_b, yp_b=yp_b):
            for u in range(_U):
                o = i * (_U * _VEC) + u * _VEC
                yt = yt_b[pl.ds(o, _VEC)]
                yp = yp_b[pl.ds(o, _VEC)]
                scaled = (yp + 1.0) * (_B / 2.0) - (yp * yt) * float(_B)
                bidx = jnp.clip(scaled.astype(jnp.int32), 0, _B - 1)
                idx = planes + bidx
                plsc.addupdate_scatter(hist, [idx], ones)
                plsc.addupdate_scatter(hist, [idx + 16 * _B], yt)
            return carry

        lax.fori_loop(0, _CHUNK // (_U * _VEC), vec_body, 0)

    def red_body(j, carry):
        acc_n = jnp.zeros((_VEC,), jnp.float32)
        acc_c = jnp.zeros((_VEC,), jnp.float32)
        for p in range(16):
            acc_n = acc_n + hist[pl.ds(p * _B + j * _VEC, _VEC)]
            acc_c = acc_c + hist[pl.ds((16 + p) * _B + j * _VEC, _VEC)]
        histr[pl.ds(j * _VEC, _VEC)] = acc_n
        histr[pl.ds(_B + j * _VEC, _VEC)] = acc_c
        return carry

    lax.fori_loop(0, _B // _VEC, red_body, 0)
    pltpu.sync_copy(histr, out_hbm.at[wid])


_sc_hist = functools.partial(
    pl.kernel,
    out_type=jax.ShapeDtypeStruct((_NW, 2 * _B), jnp.float32),
    mesh=plsc.VectorSubcoreMesh(core_axis_name="c", subcore_axis_name="s"),
    compiler_params=pltpu.CompilerParams(needs_layout_passes=False),
    scratch_types=[
        pltpu.VMEM((_CHUNK,), jnp.float32),
        pltpu.VMEM((_CHUNK,), jnp.float32),
        pltpu.VMEM((_CHUNK,), jnp.float32),
        pltpu.VMEM((_CHUNK,), jnp.float32),
        pltpu.VMEM((32 * _B,), jnp.float32),
        pltpu.VMEM((2 * _B,), jnp.float32),
        pltpu.SemaphoreType.DMA,
        pltpu.SemaphoreType.DMA,
        pltpu.SemaphoreType.DMA,
        pltpu.SemaphoreType.DMA,
    ],
)(_sc_hist_body)


_ROWS = 4096
_COLS = 1024
_BLK = 512


def _bce_body(yt_ref, yp_ref, acc_ref):
    yt = yt_ref[...]
    yp = yp_ref[...]
    logp = jnp.maximum(jnp.log(yp), -100.0)
    logq = jnp.maximum(jnp.log(1.0 - yp), -100.0)
    s = jnp.sum(yt * logp + logq - yt * logq)

    @pl.when(pl.program_id(0) == 0)
    def _init():
        acc_ref[0, 0] = 0.0

    acc_ref[0, 0] += s


def _bce_sum(yt2d, yp2d):
    return pl.pallas_call(
        _bce_body,
        grid=(_ROWS // _BLK,),
        in_specs=[
            pl.BlockSpec((_BLK, _COLS), lambda i: (i, 0)),
            pl.BlockSpec((_BLK, _COLS), lambda i: (i, 0)),
        ],
        out_specs=pl.BlockSpec(memory_space=pltpu.SMEM),
        out_shape=jax.ShapeDtypeStruct((1, 1), jnp.float32),
    )(yt2d, yp2d)


def _finish_body(hist_ref, bce_ref, out_ref):
    h = hist_ref[...].astype(jnp.float32)
    hn = jnp.sum(h[:, :_B], axis=0, keepdims=True)
    hc = jnp.sum(h[:, _B:], axis=0, keepdims=True)
    s1 = jnp.sum(hc)
    s2 = float(_N) - s1
    row = lax.broadcasted_iota(jnp.int32, (_B, _B), 0)
    col = lax.broadcasted_iota(jnp.int32, (_B, _B), 1)
    tri = (row >= col).astype(jnp.float32)
    ncum = jnp.dot(hn, tri, preferred_element_type=jnp.float32,
                   precision=lax.Precision.HIGHEST)
    c1 = jnp.dot(hc, tri, preferred_element_type=jnp.float32,
                 precision=lax.Precision.HIGHEST)
    c2 = ncum - c1

    def jsum(s, c):
        inter = s - c
        union = s + ncum - c
        iz = (union == 0.0).astype(jnp.float32)
        return jnp.sum(1.0 - (inter + iz) / jnp.maximum(union, 1.0))

    loss1 = _W * jsum(s1, c1) - _W * 0.5
    loss2 = _W * jsum(s2, c2) - _W * 0.5
    bce = -bce_ref[0, 0] / float(_N)
    out_ref[0, 0] = (loss1 + loss2) * 0.5 + bce


def _finish(hist, bce):
    return pl.pallas_call(
        _finish_body,
        in_specs=[
            pl.BlockSpec(memory_space=pltpu.VMEM),
            pl.BlockSpec(memory_space=pltpu.SMEM),
        ],
        out_specs=pl.BlockSpec(memory_space=pltpu.SMEM),
        out_shape=jax.ShapeDtypeStruct((1, 1), jnp.float32),
    )(hist, bce)


def kernel(y_true, y_pred):
    yt = y_true.reshape(-1)
    yp = y_pred.reshape(-1)
    hist = _sc_hist(yt, yp)
    bce = _bce_sum(y_true.reshape(_ROWS, _COLS), y_pred.reshape(_ROWS, _COLS))
    out = _finish(hist, bce)
    return out[0, 0]


# speedup vs baseline: 138.3891x; 1.8851x over previous
import functools

import jax
import jax.numpy as jnp
from jax import lax
from jax.experimental import pallas as pl
from jax.experimental.pallas import tpu as pltpu
from jax.experimental.pallas import tpu_sc as plsc

_B = 2048
_W = 2.0 / _B
_N = 16 * 512 * 512
_NW = 32
_PER_TILE = _N // _NW
_CHUNK = 8192
_NCHUNK = _PER_TILE // _CHUNK
_VEC = 16
_U = 8


def _sc_hist_body(yt_hbm, yp_hbm, out_hbm,
                  yt0, yp0, yt1, yp1, hist, histr, sem0, sem1, sem2, sem3):
    cid = lax.axis_index("c")
    sid = lax.axis_index("s")
    wid = sid * 2 + cid
    base = wid * _PER_TILE

    bufs = [(yt0, yp0, sem0, sem1), (yt1, yp1, sem2, sem3)]

    def copies(k, slot):
        off = base + k * _CHUNK
        yt_b, yp_b, sa, sb = bufs[slot]
        return (pltpu.make_async_copy(yt_hbm.at[pl.ds(off, _CHUNK)], yt_b, sa),
                pltpu.make_async_copy(yp_hbm.at[pl.ds(off, _CHUNK)], yp_b, sb))

    for cp in copies(0, 0):
        cp.start()

    @plsc.parallel_loop(0, (32 * _B) // _VEC, unroll=8)
    def _zero(i):
        hist[pl.ds(i * _VEC, _VEC)] = jnp.zeros((_VEC,), jnp.float32)

    planes = lax.iota(jnp.int32, _VEC) * _B
    ones = jnp.ones((_VEC,), jnp.float32)

    for k in range(_NCHUNK):
        slot = k % 2
        if k + 1 < _NCHUNK:
            for cp in copies(k + 1, 1 - slot):
                cp.start()
        for cp in copies(k, slot):
            cp.wait()
        yt_b, yp_b = bufs[slot][0], bufs[slot][1]

        @plsc.parallel_loop(0, _CHUNK // _VEC, unroll=_U)
        def _vec(i, yt_b=yt_b, yp_b=yp_b):
            o = i * _VEC
            yt = yt_b[pl.ds(o, _VEC)]
            yp = yp_b[pl.ds(o, _VEC)]
            scaled = (yp + 1.0) * (_B / 2.0) - (yp * yt) * float(_B)
            bidx = jnp.clip(scaled.astype(jnp.int32), 0, _B - 1)
            idx = planes + bidx
            plsc.addupdate_scatter(hist, [idx], ones)
            plsc.addupdate_scatter(hist, [idx + 16 * _B], yt)

    @plsc.parallel_loop(0, _B // _VEC, unroll=2)
    def _red(j):
        acc_n = jnp.zeros((_VEC,), jnp.float32)
        acc_c = jnp.zeros((_VEC,), jnp.float32)
        for p in range(16):
            acc_n = acc_n + hist[pl.ds(p * _B + j * _VEC, _VEC)]
            acc_c = acc_c + hist[pl.ds((16 + p) * _B + j * _VEC, _VEC)]
        histr[pl.ds(j * _VEC, _VEC)] = acc_n
        histr[pl.ds(_B + j * _VEC, _VEC)] = acc_c
    pltpu.sync_copy(histr, out_hbm.at[wid])


_sc_hist = functools.partial(
    pl.kernel,
    out_type=jax.ShapeDtypeStruct((_NW, 2 * _B), jnp.float32),
    mesh=plsc.VectorSubcoreMesh(core_axis_name="c", subcore_axis_name="s"),
    compiler_params=pltpu.CompilerParams(needs_layout_passes=False),
    scratch_types=[
        pltpu.VMEM((_CHUNK,), jnp.float32),
        pltpu.VMEM((_CHUNK,), jnp.float32),
        pltpu.VMEM((_CHUNK,), jnp.float32),
        pltpu.VMEM((_CHUNK,), jnp.float32),
        pltpu.VMEM((32 * _B,), jnp.float32),
        pltpu.VMEM((2 * _B,), jnp.float32),
        pltpu.SemaphoreType.DMA,
        pltpu.SemaphoreType.DMA,
        pltpu.SemaphoreType.DMA,
        pltpu.SemaphoreType.DMA,
    ],
)(_sc_hist_body)


_ROWS = 4096
_COLS = 1024
_BLK = 512


def _bce_body(yt_ref, yp_ref, acc_ref):
    yt = yt_ref[...]
    yp = yp_ref[...]
    logp = jnp.maximum(jnp.log(yp), -100.0)
    logq = jnp.maximum(jnp.log(1.0 - yp), -100.0)
    s = jnp.sum(yt * logp + logq - yt * logq)

    @pl.when(pl.program_id(0) == 0)
    def _init():
        acc_ref[0, 0] = 0.0

    acc_ref[0, 0] += s


def _bce_sum(yt2d, yp2d):
    return pl.pallas_call(
        _bce_body,
        grid=(_ROWS // _BLK,),
        in_specs=[
            pl.BlockSpec((_BLK, _COLS), lambda i: (i, 0)),
            pl.BlockSpec((_BLK, _COLS), lambda i: (i, 0)),
        ],
        out_specs=pl.BlockSpec(memory_space=pltpu.SMEM),
        out_shape=jax.ShapeDtypeStruct((1, 1), jnp.float32),
    )(yt2d, yp2d)


def _finish_body(hist_ref, bce_ref, out_ref):
    h = hist_ref[...].astype(jnp.float32)
    hn = jnp.sum(h[:, :_B], axis=0, keepdims=True)
    hc = jnp.sum(h[:, _B:], axis=0, keepdims=True)
    s1 = jnp.sum(hc)
    s2 = float(_N) - s1
    row = lax.broadcasted_iota(jnp.int32, (_B, _B), 0)
    col = lax.broadcasted_iota(jnp.int32, (_B, _B), 1)
    tri = (row >= col).astype(jnp.float32)
    ncum = jnp.dot(hn, tri, preferred_element_type=jnp.float32,
                   precision=lax.Precision.HIGHEST)
    c1 = jnp.dot(hc, tri, preferred_element_type=jnp.float32,
                 precision=lax.Precision.HIGHEST)
    c2 = ncum - c1

    def jsum(s, c):
        inter = s - c
        union = s + ncum - c
        iz = (union == 0.0).astype(jnp.float32)
        return jnp.sum(1.0 - (inter + iz) / jnp.maximum(union, 1.0))

    loss1 = _W * jsum(s1, c1) - _W * 0.5
    loss2 = _W * jsum(s2, c2) - _W * 0.5
    bce = -bce_ref[0, 0] / float(_N)
    out_ref[0, 0] = (loss1 + loss2) * 0.5 + bce


def _finish(hist, bce):
    return pl.pallas_call(
        _finish_body,
        in_specs=[
            pl.BlockSpec(memory_space=pltpu.VMEM),
            pl.BlockSpec(memory_space=pltpu.SMEM),
        ],
        out_specs=pl.BlockSpec(memory_space=pltpu.SMEM),
        out_shape=jax.ShapeDtypeStruct((1, 1), jnp.float32),
    )(hist, bce)


def kernel(y_true, y_pred):
    yt = y_true.reshape(-1)
    yp = y_pred.reshape(-1)
    hist = _sc_hist(yt, yp)
    bce = _bce_sum(y_true.reshape(_ROWS, _COLS), y_pred.reshape(_ROWS, _COLS))
    out = _finish(hist, bce)
    return out[0, 0]


# speedup vs baseline: 138.6812x; 1.0021x over previous
import functools

import jax
import jax.numpy as jnp
from jax import lax
from jax.experimental import pallas as pl
from jax.experimental.pallas import tpu as pltpu
from jax.experimental.pallas import tpu_sc as plsc

_B = 2048
_W = 2.0 / _B
_N = 16 * 512 * 512
_NW = 32
_PER_TILE = _N // _NW
_CHUNK = 8192
_NCHUNK = _PER_TILE // _CHUNK
_VEC = 16
_U = 8


def _sc_hist_body(yt_hbm, yp_hbm, out_hbm,
                  yt0, yp0, yt1, yp1, hist, histr, sem0, sem1, sem2, sem3):
    cid = lax.axis_index("c")
    sid = lax.axis_index("s")
    wid = sid * 2 + cid
    base = wid * _PER_TILE

    bufs = [(yt0, yp0, sem0, sem1), (yt1, yp1, sem2, sem3)]

    def copies(k, slot):
        off = base + k * _CHUNK
        yt_b, yp_b, sa, sb = bufs[slot]
        return (pltpu.make_async_copy(yt_hbm.at[pl.ds(off, _CHUNK)], yt_b, sa),
                pltpu.make_async_copy(yp_hbm.at[pl.ds(off, _CHUNK)], yp_b, sb))

    for cp in copies(0, 0):
        cp.start()

    @plsc.parallel_loop(0, (32 * _B) // _VEC, unroll=8)
    def _zero(i):
        hist[pl.ds(i * _VEC, _VEC)] = jnp.zeros((_VEC,), jnp.float32)

    planes = lax.iota(jnp.int32, _VEC) * _B
    ones = jnp.ones((_VEC,), jnp.float32)

    for k in range(_NCHUNK):
        slot = k % 2
        if k + 1 < _NCHUNK:
            for cp in copies(k + 1, 1 - slot):
                cp.start()
        for cp in copies(k, slot):
            cp.wait()
        yt_b, yp_b = bufs[slot][0], bufs[slot][1]

        @plsc.parallel_loop(0, _CHUNK // _VEC, unroll=_U)
        def _vec(i, yt_b=yt_b, yp_b=yp_b):
            o = i * _VEC
            yt = yt_b[pl.ds(o, _VEC)]
            yp = yp_b[pl.ds(o, _VEC)]
            scaled = (yp + 1.0) * (_B / 2.0) - (yp * yt) * float(_B)
            bidx = jnp.clip(scaled.astype(jnp.int32), 0, _B - 1)
            idx = planes + bidx
            plsc.addupdate_scatter(hist, [idx], ones)
            plsc.addupdate_scatter(hist, [idx + 16 * _B], yt)

    @plsc.parallel_loop(0, _B // _VEC, unroll=2)
    def _red(j):
        acc_n = jnp.zeros((_VEC,), jnp.float32)
        acc_c = jnp.zeros((_VEC,), jnp.float32)
        for p in range(16):
            acc_n = acc_n + hist[pl.ds(p * _B + j * _VEC, _VEC)]
            acc_c = acc_c + hist[pl.ds((16 + p) * _B + j * _VEC, _VEC)]
        histr[pl.ds(j * _VEC, _VEC)] = acc_n
        histr[pl.ds(_B + j * _VEC, _VEC)] = acc_c
    pltpu.sync_copy(histr, out_hbm.at[wid])


_sc_hist = functools.partial(
    pl.kernel,
    out_type=jax.ShapeDtypeStruct((_NW, 2 * _B), jnp.float32),
    mesh=plsc.VectorSubcoreMesh(core_axis_name="c", subcore_axis_name="s"),
    compiler_params=pltpu.CompilerParams(needs_layout_passes=False,
                                         use_tc_tiling_on_sc=True),
    scratch_types=[
        pltpu.VMEM((_CHUNK,), jnp.float32),
        pltpu.VMEM((_CHUNK,), jnp.float32),
        pltpu.VMEM((_CHUNK,), jnp.float32),
        pltpu.VMEM((_CHUNK,), jnp.float32),
        pltpu.VMEM((32 * _B,), jnp.float32),
        pltpu.VMEM((2 * _B,), jnp.float32),
        pltpu.SemaphoreType.DMA,
        pltpu.SemaphoreType.DMA,
        pltpu.SemaphoreType.DMA,
        pltpu.SemaphoreType.DMA,
    ],
)(_sc_hist_body)


_ROWS = 4096
_COLS = 1024
_BLK = 512


def _bce_body(yt_ref, yp_ref, acc_ref):
    yt = yt_ref[...]
    yp = yp_ref[...]
    logp = jnp.maximum(jnp.log(yp), -100.0)
    logq = jnp.maximum(jnp.log(1.0 - yp), -100.0)
    s = jnp.sum(yt * logp + logq - yt * logq)

    @pl.when(pl.program_id(0) == 0)
    def _init():
        acc_ref[0, 0] = 0.0

    acc_ref[0, 0] += s


def _bce_sum(yt2d, yp2d):
    return pl.pallas_call(
        _bce_body,
        grid=(_ROWS // _BLK,),
        in_specs=[
            pl.BlockSpec((_BLK, _COLS), lambda i: (i, 0)),
            pl.BlockSpec((_BLK, _COLS), lambda i: (i, 0)),
        ],
        out_specs=pl.BlockSpec(memory_space=pltpu.SMEM),
        out_shape=jax.ShapeDtypeStruct((1, 1), jnp.float32),
    )(yt2d, yp2d)


def _finish_body(hist_ref, bce_ref, out_ref):
    h = hist_ref[...].astype(jnp.float32)
    hn = jnp.sum(h[:, :_B], axis=0, keepdims=True)
    hc = jnp.sum(h[:, _B:], axis=0, keepdims=True)
    s1 = jnp.sum(hc)
    s2 = float(_N) - s1
    row = lax.broadcasted_iota(jnp.int32, (_B, _B), 0)
    col = lax.broadcasted_iota(jnp.int32, (_B, _B), 1)
    tri = (row >= col).astype(jnp.float32)
    ncum = jnp.dot(hn, tri, preferred_element_type=jnp.float32,
                   precision=lax.Precision.HIGHEST)
    c1 = jnp.dot(hc, tri, preferred_element_type=jnp.float32,
                 precision=lax.Precision.HIGHEST)
    c2 = ncum - c1

    def jsum(s, c):
        inter = s - c
        union = s + ncum - c
        iz = (union == 0.0).astype(jnp.float32)
        return jnp.sum(1.0 - (inter + iz) / jnp.maximum(union, 1.0))

    loss1 = _W * jsum(s1, c1) - _W * 0.5
    loss2 = _W * jsum(s2, c2) - _W * 0.5
    bce = -bce_ref[0, 0] / float(_N)
    out_ref[0, 0] = (loss1 + loss2) * 0.5 + bce


def _finish(hist, bce):
    return pl.pallas_call(
        _finish_body,
        in_specs=[
            pl.BlockSpec(memory_space=pltpu.VMEM),
            pl.BlockSpec(memory_space=pltpu.SMEM),
        ],
        out_specs=pl.BlockSpec(memory_space=pltpu.SMEM),
        out_shape=jax.ShapeDtypeStruct((1, 1), jnp.float32),
    )(hist, bce)


def kernel(y_true, y_pred):
    yt = y_true.reshape(-1)
    yp = y_pred.reshape(-1)
    hist = _sc_hist(yt, yp)
    bce = _bce_sum(y_true.reshape(_ROWS, _COLS), y_pred.reshape(_ROWS, _COLS))
    out = _finish(hist, bce)
    return out[0, 0]


# speedup vs baseline: 169.1809x; 1.2199x over previous
import functools

import jax
import jax.numpy as jnp
from jax import lax
from jax.experimental import pallas as pl
from jax.experimental.pallas import tpu as pltpu
from jax.experimental.pallas import tpu_sc as plsc

_B = 2048
_W = 2.0 / _B
_N = 16 * 512 * 512
_NW = 32
_PER_TILE = _N // _NW
_CHUNK = 8192
_NCHUNK = _PER_TILE // _CHUNK
_VEC = 16
_U = 8


def _sc_hist_body(yt_hbm, yp_hbm, out_hbm,
                  yt0, yp0, yt1, yp1, hist, histr, sem0, sem1, sem2, sem3):
    cid = lax.axis_index("c")
    sid = lax.axis_index("s")
    wid = sid * 2 + cid
    base = wid * _PER_TILE

    bufs = [(yt0, yp0, sem0, sem1), (yt1, yp1, sem2, sem3)]

    def copies(k, slot):
        off = base + k * _CHUNK
        yt_b, yp_b, sa, sb = bufs[slot]
        return (pltpu.make_async_copy(yt_hbm.at[pl.ds(off, _CHUNK)], yt_b, sa),
                pltpu.make_async_copy(yp_hbm.at[pl.ds(off, _CHUNK)], yp_b, sb))

    for cp in copies(0, 0):
        cp.start()

    @plsc.parallel_loop(0, (16 * _B) // _VEC, unroll=8)
    def _zero(i):
        hist[pl.ds(i * _VEC, _VEC)] = jnp.zeros((_VEC,), jnp.float32)

    planes = lax.iota(jnp.int32, _VEC) * _B
    ones = jnp.ones((_VEC,), jnp.float32)

    for k in range(_NCHUNK):
        slot = k % 2
        if k + 1 < _NCHUNK:
            for cp in copies(k + 1, 1 - slot):
                cp.start()
        for cp in copies(k, slot):
            cp.wait()
        yt_b, yp_b = bufs[slot][0], bufs[slot][1]

        @plsc.parallel_loop(0, _CHUNK // _VEC, unroll=_U)
        def _vec(i, yt_b=yt_b, yp_b=yp_b):
            o = i * _VEC
            yt = yt_b[pl.ds(o, _VEC)]
            yp = yp_b[pl.ds(o, _VEC)]
            scaled = (yp + 1.0) * (_B / 2.0) - yt * (yp * float(_B) + 0.5)
            bidx = jnp.clip(scaled.astype(jnp.int32), 0, _B - 1)
            plsc.addupdate_scatter(hist, [planes + bidx], ones)

    @plsc.parallel_loop(0, _B // _VEC, unroll=2)
    def _red(j):
        acc_n = jnp.zeros((_VEC,), jnp.float32)
        for p in range(16):
            acc_n = acc_n + hist[pl.ds(p * _B + j * _VEC, _VEC)]
        histr[pl.ds(j * _VEC, _VEC)] = acc_n
    pltpu.sync_copy(histr, out_hbm.at[wid])


_sc_hist = functools.partial(
    pl.kernel,
    out_type=jax.ShapeDtypeStruct((_NW, _B), jnp.float32),
    mesh=plsc.VectorSubcoreMesh(core_axis_name="c", subcore_axis_name="s"),
    compiler_params=pltpu.CompilerParams(needs_layout_passes=False,
                                         use_tc_tiling_on_sc=True),
    scratch_types=[
        pltpu.VMEM((_CHUNK,), jnp.float32),
        pltpu.VMEM((_CHUNK,), jnp.float32),
        pltpu.VMEM((_CHUNK,), jnp.float32),
        pltpu.VMEM((_CHUNK,), jnp.float32),
        pltpu.VMEM((16 * _B,), jnp.float32),
        pltpu.VMEM((_B,), jnp.float32),
        pltpu.SemaphoreType.DMA,
        pltpu.SemaphoreType.DMA,
        pltpu.SemaphoreType.DMA,
        pltpu.SemaphoreType.DMA,
    ],
)(_sc_hist_body)


def _bce_body(yt_ref, yp_ref, acc_ref):
    yt = yt_ref[...]
    yp = yp_ref[...]
    logp = jnp.maximum(jnp.log(yp), -100.0)
    logq = jnp.maximum(jnp.log(1.0 - yp), -100.0)
    s = jnp.sum(yt * logp + logq - yt * logq)

    @pl.when(pl.program_id(0) == 0)
    def _init():
        acc_ref[0, 0] = 0.0

    acc_ref[0, 0] += s


def _bce_sum(yt3d, yp3d):
    return pl.pallas_call(
        _bce_body,
        grid=(8,),
        in_specs=[
            pl.BlockSpec((2, 512, 512), lambda i: (i, 0, 0)),
            pl.BlockSpec((2, 512, 512), lambda i: (i, 0, 0)),
        ],
        out_specs=pl.BlockSpec(memory_space=pltpu.SMEM),
        out_shape=jax.ShapeDtypeStruct((1, 1), jnp.float32),
    )(yt3d, yp3d)


def _finish_body(hist_ref, bce_ref, out_ref):
    h = hist_ref[...]
    hn = jnp.sum(h, axis=0, keepdims=True)
    pos = (lax.broadcasted_iota(jnp.int32, (1, _B), 1) < _B // 2)
    hc = jnp.where(pos, hn, 0.0)
    s1 = jnp.sum(hc)
    s2 = float(_N) - s1
    row = lax.broadcasted_iota(jnp.int32, (_B, _B), 0)
    col = lax.broadcasted_iota(jnp.int32, (_B, _B), 1)
    tri = (row >= col).astype(jnp.float32)
    ncum = jnp.dot(hn, tri, preferred_element_type=jnp.float32,
                   precision=lax.Precision.HIGHEST)
    c1 = jnp.dot(hc, tri, preferred_element_type=jnp.float32,
                 precision=lax.Precision.HIGHEST)
    c2 = ncum - c1

    def jsum(s, c):
        inter = s - c
        union = s + ncum - c
        iz = (union == 0.0).astype(jnp.float32)
        return jnp.sum(1.0 - (inter + iz) / jnp.maximum(union, 1.0))

    loss1 = _W * jsum(s1, c1) - _W * 0.5
    loss2 = _W * jsum(s2, c2) - _W * 0.5
    bce = -bce_ref[0, 0] / float(_N)
    out_ref[0, 0] = (loss1 + loss2) * 0.5 + bce


def _finish(hist, bce):
    return pl.pallas_call(
        _finish_body,
        in_specs=[
            pl.BlockSpec(memory_space=pltpu.VMEM),
            pl.BlockSpec(memory_space=pltpu.SMEM),
        ],
        out_specs=pl.BlockSpec(memory_space=pltpu.SMEM),
        out_shape=jax.ShapeDtypeStruct((1, 1), jnp.float32),
    )(hist, bce)


def kernel(y_true, y_pred):
    yt = y_true.reshape(-1)
    yp = y_pred.reshape(-1)
    hist = _sc_hist(yt, yp)
    bce = _bce_sum(y_true, y_pred)
    out = _finish(hist, bce)
    return out[0, 0]


# speedup vs baseline: 174.7799x; 1.0331x over previous
import functools

import jax
import jax.numpy as jnp
from jax import lax
from jax.experimental import pallas as pl
from jax.experimental.pallas import tpu as pltpu
from jax.experimental.pallas import tpu_sc as plsc

_B = 2048
_W = 2.0 / _B
_N = 16 * 512 * 512
_NW = 32
_PER_TILE = _N // _NW
_CHUNK = 8192
_NCHUNK = _PER_TILE // _CHUNK
_VEC = 16
_U = 8


def _sc_hist_body(yt_hbm, yp_hbm, out_hbm,
                  yt0, yp0, yt1, yp1, hist, histr, sem0, sem1, sem2, sem3):
    cid = lax.axis_index("c")
    sid = lax.axis_index("s")
    wid = sid * 2 + cid
    base = wid * _PER_TILE

    bufs = [(yt0, yp0, sem0, sem1), (yt1, yp1, sem2, sem3)]

    def copies(k, slot):
        off = base + k * _CHUNK
        yt_b, yp_b, sa, sb = bufs[slot]
        return (pltpu.make_async_copy(yt_hbm.at[pl.ds(off, _CHUNK)], yt_b, sa),
                pltpu.make_async_copy(yp_hbm.at[pl.ds(off, _CHUNK)], yp_b, sb))

    for cp in copies(0, 0):
        cp.start()

    @plsc.parallel_loop(0, (16 * _B) // _VEC, unroll=8)
    def _zero(i):
        hist[pl.ds(i * _VEC, _VEC)] = jnp.zeros((_VEC,), jnp.float32)

    planes = lax.iota(jnp.int32, _VEC) * _B
    ones = jnp.ones((_VEC,), jnp.float32)

    for k in range(_NCHUNK):
        slot = k % 2
        if k + 1 < _NCHUNK:
            for cp in copies(k + 1, 1 - slot):
                cp.start()
        for cp in copies(k, slot):
            cp.wait()
        yt_b, yp_b = bufs[slot][0], bufs[slot][1]

        @plsc.parallel_loop(0, _CHUNK // _VEC, unroll=_U)
        def _vec(i, yt_b=yt_b, yp_b=yp_b):
            o = i * _VEC
            yt = yt_b[pl.ds(o, _VEC)]
            yp = yp_b[pl.ds(o, _VEC)]
            scaled = (yp + 1.0) * (_B / 2.0) - yt * (yp * float(_B) + 0.5)
            bidx = jnp.clip(scaled.astype(jnp.int32), 0, _B - 1)
            plsc.addupdate_scatter(hist, [planes + bidx], ones)

    @plsc.parallel_loop(0, _B // _VEC, unroll=2)
    def _red(j):
        acc_n = jnp.zeros((_VEC,), jnp.float32)
        for p in range(16):
            acc_n = acc_n + hist[pl.ds(p * _B + j * _VEC, _VEC)]
        histr[pl.ds(j * _VEC, _VEC)] = acc_n
    pltpu.sync_copy(histr, out_hbm.at[wid])


_sc_hist = functools.partial(
    pl.kernel,
    out_type=jax.ShapeDtypeStruct((_NW, _B), jnp.float32),
    mesh=plsc.VectorSubcoreMesh(core_axis_name="c", subcore_axis_name="s"),
    compiler_params=pltpu.CompilerParams(needs_layout_passes=False,
                                         use_tc_tiling_on_sc=True),
    scratch_types=[
        pltpu.VMEM((_CHUNK,), jnp.float32),
        pltpu.VMEM((_CHUNK,), jnp.float32),
        pltpu.VMEM((_CHUNK,), jnp.float32),
        pltpu.VMEM((_CHUNK,), jnp.float32),
        pltpu.VMEM((16 * _B,), jnp.float32),
        pltpu.VMEM((_B,), jnp.float32),
        pltpu.SemaphoreType.DMA,
        pltpu.SemaphoreType.DMA,
        pltpu.SemaphoreType.DMA,
        pltpu.SemaphoreType.DMA,
    ],
)(_sc_hist_body)


def _bce_body(yt_ref, yp_ref, acc_ref):
    yt = yt_ref[...]
    yp = yp_ref[...]
    logp = jnp.maximum(jnp.log(yp), -100.0)
    logq = jnp.maximum(jnp.log(1.0 - yp), -100.0)
    s = jnp.sum(yt * logp + logq - yt * logq)

    @pl.when(pl.program_id(0) == 0)
    def _init():
        acc_ref[0, 0] = 0.0

    acc_ref[0, 0] += s


def _bce_sum(yt3d, yp3d):
    return pl.pallas_call(
        _bce_body,
        grid=(8,),
        in_specs=[
            pl.BlockSpec((2, 512, 512), lambda i: (i, 0, 0)),
            pl.BlockSpec((2, 512, 512), lambda i: (i, 0, 0)),
        ],
        out_specs=pl.BlockSpec(memory_space=pltpu.SMEM),
        out_shape=jax.ShapeDtypeStruct((1, 1), jnp.float32),
    )(yt3d, yp3d)


def _finish_body(hist_ref, bce_ref, out_ref):
    h = hist_ref[...]
    hn = jnp.sum(h, axis=0, keepdims=True)
    pos = (lax.broadcasted_iota(jnp.int32, (1, _B), 1) < _B // 2)
    hc = jnp.where(pos, hn, 0.0)
    s1 = jnp.sum(hc)
    s2 = float(_N) - s1
    row = lax.broadcasted_iota(jnp.int32, (_B, _B), 0)
    col = lax.broadcasted_iota(jnp.int32, (_B, _B), 1)
    tri = (row >= col).astype(jnp.float32)
    hnc = jnp.concatenate([hn, hc], axis=0)
    cum = jnp.dot(hnc, tri, preferred_element_type=jnp.float32,
                  precision=lax.Precision.HIGHEST)
    ncum = cum[0:1, :]
    c1 = cum[1:2, :]
    c2 = ncum - c1

    def jsum(s, c):
        inter = s - c
        union = s + ncum - c
        iz = (union == 0.0).astype(jnp.float32)
        return jnp.sum(1.0 - (inter + iz) / jnp.maximum(union, 1.0))

    loss1 = _W * jsum(s1, c1) - _W * 0.5
    loss2 = _W * jsum(s2, c2) - _W * 0.5
    bce = -bce_ref[0, 0] / float(_N)
    out_ref[0, 0] = (loss1 + loss2) * 0.5 + bce


def _finish(hist, bce):
    return pl.pallas_call(
        _finish_body,
        in_specs=[
            pl.BlockSpec(memory_space=pltpu.VMEM),
            pl.BlockSpec(memory_space=pltpu.SMEM),
        ],
        out_specs=pl.BlockSpec(memory_space=pltpu.SMEM),
        out_shape=jax.ShapeDtypeStruct((1, 1), jnp.float32),
    )(hist, bce)


def kernel(y_true, y_pred):
    yt = y_true.reshape(-1)
    yp = y_pred.reshape(-1)
    bce = _bce_sum(y_true, y_pred)
    hist = _sc_hist(yt, yp)
    out = _finish(hist, bce)
    return out[0, 0]


# speedup vs baseline: 288.3968x; 1.6501x over previous
import functools

import jax
import jax.numpy as jnp
from jax import lax
from jax.experimental import pallas as pl
from jax.experimental.pallas import tpu as pltpu
from jax.experimental.pallas import tpu_sc as plsc

_B = 2048
_W = 2.0 / _B
_N = 16 * 512 * 512
_NW = 32
_PER_TILE = _N // _NW
_CHUNK = 16384
_NCHUNK = _PER_TILE // _CHUNK
_VEC = 16
_U = 8


_CROWS = 32


def _sc_hist_body(yt_hbm, yp_hbm, out_hbm,
                  yt0, yp0, yt1, yp1, hist, histr, sem0, sem1, sem2, sem3):
    cid = lax.axis_index("c")
    sid = lax.axis_index("s")
    wid = sid * 2 + cid
    batch = wid // 2
    row0 = (wid % 2) * 256

    bufs = [(yt0, yp0, sem0, sem1), (yt1, yp1, sem2, sem3)]

    def copies(k, slot):
        r = row0 + k * _CROWS
        yt_b, yp_b, sa, sb = bufs[slot]
        return (pltpu.make_async_copy(
                    yt_hbm.at[batch, pl.ds(r, _CROWS), :], yt_b, sa),
                pltpu.make_async_copy(
                    yp_hbm.at[batch, pl.ds(r, _CROWS), :], yp_b, sb))

    for cp in copies(0, 0):
        cp.start()

    @plsc.parallel_loop(0, (16 * _B) // _VEC, unroll=8)
    def _zero(i):
        hist[pl.ds(i * _VEC, _VEC)] = jnp.zeros((_VEC,), jnp.float32)

    planes = lax.iota(jnp.int32, _VEC) * _B
    ones = jnp.ones((_VEC,), jnp.float32)

    for k in range(_NCHUNK):
        slot = k % 2
        if k + 1 < _NCHUNK:
            for cp in copies(k + 1, 1 - slot):
                cp.start()
        for cp in copies(k, slot):
            cp.wait()
        yt_b, yp_b = bufs[slot][0], bufs[slot][1]

        @plsc.parallel_loop(0, _CROWS * (512 // _VEC), unroll=_U)
        def _vec(i, yt_b=yt_b, yp_b=yp_b):
            r = lax.shift_right_logical(i, 5)
            o = lax.shift_left(lax.bitwise_and(i, 31), 4)
            yt = yt_b[r, pl.ds(o, _VEC)]
            yp = yp_b[r, pl.ds(o, _VEC)]
            scaled = (yp + 1.0) * (_B / 2.0) - yt * (yp * float(_B) + 0.5)
            bidx = jnp.clip(scaled.astype(jnp.int32), 0, _B - 1)
            plsc.addupdate_scatter(hist, [planes + bidx], ones)

    @plsc.parallel_loop(0, _B // _VEC, unroll=2)
    def _red(j):
        acc_n = jnp.zeros((_VEC,), jnp.float32)
        for p in range(16):
            acc_n = acc_n + hist[pl.ds(p * _B + j * _VEC, _VEC)]
        histr[pl.ds(j * _VEC, _VEC)] = acc_n
    pltpu.sync_copy(histr, out_hbm.at[wid])


_sc_hist = functools.partial(
    pl.kernel,
    out_type=jax.ShapeDtypeStruct((_NW, _B), jnp.float32),
    mesh=plsc.VectorSubcoreMesh(core_axis_name="c", subcore_axis_name="s"),
    compiler_params=pltpu.CompilerParams(needs_layout_passes=False,
                                         use_tc_tiling_on_sc=True),
    scratch_types=[
        pltpu.VMEM((_CROWS, 512), jnp.float32),
        pltpu.VMEM((_CROWS, 512), jnp.float32),
        pltpu.VMEM((_CROWS, 512), jnp.float32),
        pltpu.VMEM((_CROWS, 512), jnp.float32),
        pltpu.VMEM((16 * _B,), jnp.float32),
        pltpu.VMEM((_B,), jnp.float32),
        pltpu.SemaphoreType.DMA,
        pltpu.SemaphoreType.DMA,
        pltpu.SemaphoreType.DMA,
        pltpu.SemaphoreType.DMA,
    ],
)(_sc_hist_body)


def _bce_body(yt_ref, yp_ref, acc_ref):
    yt = yt_ref[...]
    yp = yp_ref[...]
    logp = jnp.maximum(jnp.log(yp), -100.0)
    logq = jnp.maximum(jnp.log(1.0 - yp), -100.0)
    s = jnp.sum(yt * logp + logq - yt * logq)

    @pl.when(pl.program_id(0) == 0)
    def _init():
        acc_ref[0, 0] = 0.0

    acc_ref[0, 0] += s


def _bce_sum(yt3d, yp3d):
    return pl.pallas_call(
        _bce_body,
        grid=(8,),
        in_specs=[
            pl.BlockSpec((2, 512, 512), lambda i: (i, 0, 0)),
            pl.BlockSpec((2, 512, 512), lambda i: (i, 0, 0)),
        ],
        out_specs=pl.BlockSpec(memory_space=pltpu.SMEM),
        out_shape=jax.ShapeDtypeStruct((1, 1), jnp.float32),
    )(yt3d, yp3d)


def _finish_body(hist_ref, bce_ref, out_ref):
    h = hist_ref[...]
    hn = jnp.sum(h, axis=0, keepdims=True)
    pos = (lax.broadcasted_iota(jnp.int32, (1, _B), 1) < _B // 2)
    hc = jnp.where(pos, hn, 0.0)
    s1 = jnp.sum(hc)
    s2 = float(_N) - s1
    row = lax.broadcasted_iota(jnp.int32, (_B, _B), 0)
    col = lax.broadcasted_iota(jnp.int32, (_B, _B), 1)
    tri = (row >= col).astype(jnp.float32)
    hnc = jnp.concatenate([hn, hc], axis=0)
    cum = jnp.dot(hnc, tri, preferred_element_type=jnp.float32,
                  precision=lax.Precision.HIGHEST)
    ncum = cum[0:1, :]
    c1 = cum[1:2, :]
    c2 = ncum - c1

    def jsum(s, c):
        inter = s - c
        union = s + ncum - c
        iz = (union == 0.0).astype(jnp.float32)
        return jnp.sum(1.0 - (inter + iz) / jnp.maximum(union, 1.0))

    loss1 = _W * jsum(s1, c1) - _W * 0.5
    loss2 = _W * jsum(s2, c2) - _W * 0.5
    bce = -bce_ref[0, 0] / float(_N)
    out_ref[0, 0] = (loss1 + loss2) * 0.5 + bce


def _finish(hist, bce):
    return pl.pallas_call(
        _finish_body,
        in_specs=[
            pl.BlockSpec(memory_space=pltpu.VMEM),
            pl.BlockSpec(memory_space=pltpu.SMEM),
        ],
        out_specs=pl.BlockSpec(memory_space=pltpu.SMEM),
        out_shape=jax.ShapeDtypeStruct((1, 1), jnp.float32),
    )(hist, bce)


def kernel(y_true, y_pred):
    bce = _bce_sum(y_true, y_pred)
    hist = _sc_hist(y_true, y_pred)
    out = _finish(hist, bce)
    return out[0, 0]


# speedup vs baseline: 298.3590x; 1.0345x over previous
import functools

import jax
import jax.numpy as jnp
from jax import lax
from jax.experimental import pallas as pl
from jax.experimental.pallas import tpu as pltpu
from jax.experimental.pallas import tpu_sc as plsc

_B = 2048
_W = 2.0 / _B
_N = 16 * 512 * 512
_NW = 32
_PER_TILE = _N // _NW
_CHUNK = 16384
_NCHUNK = _PER_TILE // _CHUNK
_VEC = 16
_U = 8


_CROWS = 32


def _sc_hist_body(yt_hbm, yp_hbm, out_hbm,
                  yt0, yp0, yt1, yp1, hist, histr, sem0, sem1, sem2, sem3):
    cid = lax.axis_index("c")
    sid = lax.axis_index("s")
    wid = sid * 2 + cid
    batch = wid // 2
    row0 = (wid % 2) * 256

    bufs = [(yt0, yp0, sem0, sem1), (yt1, yp1, sem2, sem3)]

    def copies(k, slot):
        r = row0 + k * _CROWS
        yt_b, yp_b, sa, sb = bufs[slot]
        return (pltpu.make_async_copy(
                    yt_hbm.at[batch, pl.ds(r, _CROWS), :], yt_b, sa),
                pltpu.make_async_copy(
                    yp_hbm.at[batch, pl.ds(r, _CROWS), :], yp_b, sb))

    for cp in copies(0, 0):
        cp.start()

    @plsc.parallel_loop(0, (16 * _B) // _VEC, unroll=8)
    def _zero(i):
        hist[pl.ds(i * _VEC, _VEC)] = jnp.zeros((_VEC,), jnp.float32)

    planes = lax.iota(jnp.int32, _VEC) * _B
    ones = jnp.ones((_VEC,), jnp.float32)

    for k in range(_NCHUNK):
        slot = k % 2
        if k + 1 < _NCHUNK:
            for cp in copies(k + 1, 1 - slot):
                cp.start()
        for cp in copies(k, slot):
            cp.wait()
        yt_b, yp_b = bufs[slot][0], bufs[slot][1]

        @plsc.parallel_loop(0, _CROWS * (512 // _VEC), unroll=_U)
        def _vec(i, yt_b=yt_b, yp_b=yp_b):
            r = lax.shift_right_logical(i, 5)
            o = lax.shift_left(lax.bitwise_and(i, 31), 4)
            yt = yt_b[r, pl.ds(o, _VEC)]
            yp = yp_b[r, pl.ds(o, _VEC)]
            scaled = (yp + 1.0) * (_B / 2.0) - yt * (yp * float(_B) + 0.5)
            bidx = scaled.astype(jnp.int32)
            plsc.addupdate_scatter(hist, [planes + bidx], ones)

    @plsc.parallel_loop(0, _B // _VEC, unroll=2)
    def _red(j):
        acc_n = jnp.zeros((_VEC,), jnp.float32)
        for p in range(16):
            acc_n = acc_n + hist[pl.ds(p * _B + j * _VEC, _VEC)]
        histr[pl.ds(j * _VEC, _VEC)] = acc_n
    pltpu.sync_copy(histr, out_hbm.at[wid])


_sc_hist = functools.partial(
    pl.kernel,
    out_type=jax.ShapeDtypeStruct((_NW, _B), jnp.float32),
    mesh=plsc.VectorSubcoreMesh(core_axis_name="c", subcore_axis_name="s"),
    compiler_params=pltpu.CompilerParams(needs_layout_passes=False,
                                         use_tc_tiling_on_sc=True),
    scratch_types=[
        pltpu.VMEM((_CROWS, 512), jnp.float32),
        pltpu.VMEM((_CROWS, 512), jnp.float32),
        pltpu.VMEM((_CROWS, 512), jnp.float32),
        pltpu.VMEM((_CROWS, 512), jnp.float32),
        pltpu.VMEM((16 * _B,), jnp.float32),
        pltpu.VMEM((_B,), jnp.float32),
        pltpu.SemaphoreType.DMA,
        pltpu.SemaphoreType.DMA,
        pltpu.SemaphoreType.DMA,
        pltpu.SemaphoreType.DMA,
    ],
)(_sc_hist_body)


def _bce_body(yt_ref, yp_ref, acc_ref):
    yt = yt_ref[...]
    yp = yp_ref[...]
    logp = jnp.maximum(jnp.log(yp), -100.0)
    logq = jnp.maximum(jnp.log(1.0 - yp), -100.0)
    s = jnp.sum(yt * logp + logq - yt * logq)

    @pl.when(pl.program_id(0) == 0)
    def _init():
        acc_ref[0, 0] = 0.0

    acc_ref[0, 0] += s


def _bce_sum(yt3d, yp3d):
    return pl.pallas_call(
        _bce_body,
        grid=(8,),
        in_specs=[
            pl.BlockSpec((2, 512, 512), lambda i: (i, 0, 0)),
            pl.BlockSpec((2, 512, 512), lambda i: (i, 0, 0)),
        ],
        out_specs=pl.BlockSpec(memory_space=pltpu.SMEM),
        out_shape=jax.ShapeDtypeStruct((1, 1), jnp.float32),
    )(yt3d, yp3d)


def _finish_body(hist_ref, bce_ref, out_ref):
    h = hist_ref[...]
    hn = jnp.sum(h, axis=0, keepdims=True)
    pos = (lax.broadcasted_iota(jnp.int32, (1, _B), 1) < _B // 2)
    hc = jnp.where(pos, hn, 0.0)
    s1 = jnp.sum(hc)
    s2 = float(_N) - s1
    row = lax.broadcasted_iota(jnp.int32, (_B, _B), 0)
    col = lax.broadcasted_iota(jnp.int32, (_B, _B), 1)
    tri = (row >= col).astype(jnp.float32)
    hnc = jnp.concatenate([hn, hc], axis=0)
    cum = jnp.dot(hnc, tri, preferred_element_type=jnp.float32,
                  precision=lax.Precision.HIGHEST)
    ncum = cum[0:1, :]
    c1 = cum[1:2, :]
    c2 = ncum - c1

    def jsum(s, c):
        inter = s - c
        union = s + ncum - c
        iz = (union == 0.0).astype(jnp.float32)
        return jnp.sum(1.0 - (inter + iz) / jnp.maximum(union, 1.0))

    loss1 = _W * jsum(s1, c1) - _W * 0.5
    loss2 = _W * jsum(s2, c2) - _W * 0.5
    bce = -bce_ref[0, 0] / float(_N)
    out_ref[0, 0] = (loss1 + loss2) * 0.5 + bce


def _finish(hist, bce):
    return pl.pallas_call(
        _finish_body,
        in_specs=[
            pl.BlockSpec(memory_space=pltpu.VMEM),
            pl.BlockSpec(memory_space=pltpu.SMEM),
        ],
        out_specs=pl.BlockSpec(memory_space=pltpu.SMEM),
        out_shape=jax.ShapeDtypeStruct((1, 1), jnp.float32),
    )(hist, bce)


def kernel(y_true, y_pred):
    bce = _bce_sum(y_true, y_pred)
    hist = _sc_hist(y_true, y_pred)
    out = _finish(hist, bce)
    return out[0, 0]


# speedup vs baseline: 312.6116x; 1.0478x over previous
import functools

import jax
import jax.numpy as jnp
from jax import lax
from jax.experimental import pallas as pl
from jax.experimental.pallas import tpu as pltpu
from jax.experimental.pallas import tpu_sc as plsc

_B = 1024
_W = 2.0 / _B
_N = 16 * 512 * 512
_NW = 32
_PER_TILE = _N // _NW
_CHUNK = 16384
_NCHUNK = _PER_TILE // _CHUNK
_VEC = 16
_U = 8


_CROWS = 32


def _sc_hist_body(yt_hbm, yp_hbm, out_hbm,
                  yt0, yp0, yt1, yp1, hist, histr, sem0, sem1, sem2, sem3):
    cid = lax.axis_index("c")
    sid = lax.axis_index("s")
    wid = sid * 2 + cid
    batch = wid // 2
    row0 = (wid % 2) * 256

    bufs = [(yt0, yp0, sem0, sem1), (yt1, yp1, sem2, sem3)]

    def copies(k, slot):
        r = row0 + k * _CROWS
        yt_b, yp_b, sa, sb = bufs[slot]
        return (pltpu.make_async_copy(
                    yt_hbm.at[batch, pl.ds(r, _CROWS), :], yt_b, sa),
                pltpu.make_async_copy(
                    yp_hbm.at[batch, pl.ds(r, _CROWS), :], yp_b, sb))

    for cp in copies(0, 0):
        cp.start()

    @plsc.parallel_loop(0, (16 * _B) // _VEC, unroll=8)
    def _zero(i):
        hist[pl.ds(i * _VEC, _VEC)] = jnp.zeros((_VEC,), jnp.float32)

    planes = lax.iota(jnp.int32, _VEC) * _B
    ones = jnp.ones((_VEC,), jnp.float32)

    for k in range(_NCHUNK):
        slot = k % 2
        if k + 1 < _NCHUNK:
            for cp in copies(k + 1, 1 - slot):
                cp.start()
        for cp in copies(k, slot):
            cp.wait()
        yt_b, yp_b = bufs[slot][0], bufs[slot][1]

        @plsc.parallel_loop(0, _CROWS * (512 // _VEC), unroll=_U)
        def _vec(i, yt_b=yt_b, yp_b=yp_b):
            r = lax.shift_right_logical(i, 5)
            o = lax.shift_left(lax.bitwise_and(i, 31), 4)
            yt = yt_b[r, pl.ds(o, _VEC)]
            yp = yp_b[r, pl.ds(o, _VEC)]
            scaled = (yp + 1.0) * (_B / 2.0) - yt * (yp * float(_B) + 0.5)
            bidx = scaled.astype(jnp.int32)
            plsc.addupdate_scatter(hist, [planes + bidx], ones)

    @plsc.parallel_loop(0, _B // _VEC, unroll=2)
    def _red(j):
        acc_n = jnp.zeros((_VEC,), jnp.float32)
        for p in range(16):
            acc_n = acc_n + hist[pl.ds(p * _B + j * _VEC, _VEC)]
        histr[pl.ds(j * _VEC, _VEC)] = acc_n
    pltpu.sync_copy(histr, out_hbm.at[wid])


_sc_hist = functools.partial(
    pl.kernel,
    out_type=jax.ShapeDtypeStruct((_NW, _B), jnp.float32),
    mesh=plsc.VectorSubcoreMesh(core_axis_name="c", subcore_axis_name="s"),
    compiler_params=pltpu.CompilerParams(needs_layout_passes=False,
                                         use_tc_tiling_on_sc=True),
    scratch_types=[
        pltpu.VMEM((_CROWS, 512), jnp.float32),
        pltpu.VMEM((_CROWS, 512), jnp.float32),
        pltpu.VMEM((_CROWS, 512), jnp.float32),
        pltpu.VMEM((_CROWS, 512), jnp.float32),
        pltpu.VMEM((16 * _B,), jnp.float32),
        pltpu.VMEM((_B,), jnp.float32),
        pltpu.SemaphoreType.DMA,
        pltpu.SemaphoreType.DMA,
        pltpu.SemaphoreType.DMA,
        pltpu.SemaphoreType.DMA,
    ],
)(_sc_hist_body)


def _bce_body(yt_ref, yp_ref, acc_ref):
    yt = yt_ref[...]
    yp = yp_ref[...]
    logp = jnp.maximum(jnp.log(yp), -100.0)
    logq = jnp.maximum(jnp.log(1.0 - yp), -100.0)
    s = jnp.sum(yt * logp + logq - yt * logq)

    @pl.when(pl.program_id(0) == 0)
    def _init():
        acc_ref[0, 0] = 0.0

    acc_ref[0, 0] += s


def _bce_sum(yt3d, yp3d):
    return pl.pallas_call(
        _bce_body,
        grid=(8,),
        in_specs=[
            pl.BlockSpec((2, 512, 512), lambda i: (i, 0, 0)),
            pl.BlockSpec((2, 512, 512), lambda i: (i, 0, 0)),
        ],
        out_specs=pl.BlockSpec(memory_space=pltpu.SMEM),
        out_shape=jax.ShapeDtypeStruct((1, 1), jnp.float32),
    )(yt3d, yp3d)


def _finish_body(hist_ref, bce_ref, out_ref):
    h = hist_ref[...]
    hn = jnp.sum(h, axis=0, keepdims=True)
    pos = (lax.broadcasted_iota(jnp.int32, (1, _B), 1) < _B // 2)
    hc = jnp.where(pos, hn, 0.0)
    s1 = jnp.sum(hc)
    s2 = float(_N) - s1
    row = lax.broadcasted_iota(jnp.int32, (_B, _B), 0)
    col = lax.broadcasted_iota(jnp.int32, (_B, _B), 1)
    tri = (row >= col).astype(jnp.float32)
    hnc = jnp.concatenate([hn, hc], axis=0)
    cum = jnp.dot(hnc, tri, preferred_element_type=jnp.float32,
                  precision=lax.Precision.HIGHEST)
    ncum = cum[0:1, :]
    c1 = cum[1:2, :]
    c2 = ncum - c1

    mid = (lax.broadcasted_iota(jnp.int32, (1, _B), 1) == _B // 2)

    def jloss(s, c):
        inter = s - c
        union = s + ncum - c
        iz = (union == 0.0).astype(jnp.float32)
        j = 1.0 - (inter + iz) / jnp.maximum(union, 1.0)
        return _W * jnp.sum(j) - _W * 0.5 * jnp.sum(jnp.where(mid, j, 0.0))

    bce = -bce_ref[0, 0] / float(_N)
    out_ref[0, 0] = (jloss(s1, c1) + jloss(s2, c2)) * 0.5 + bce


def _finish(hist, bce):
    return pl.pallas_call(
        _finish_body,
        in_specs=[
            pl.BlockSpec(memory_space=pltpu.VMEM),
            pl.BlockSpec(memory_space=pltpu.SMEM),
        ],
        out_specs=pl.BlockSpec(memory_space=pltpu.SMEM),
        out_shape=jax.ShapeDtypeStruct((1, 1), jnp.float32),
    )(hist, bce)


def kernel(y_true, y_pred):
    bce = _bce_sum(y_true, y_pred)
    hist = _sc_hist(y_true, y_pred)
    out = _finish(hist, bce)
    return out[0, 0]
